# baseline (speedup 1.0000x reference)
import functools

import jax
import jax.numpy as jnp
from jax import lax
from jax.experimental import pallas as pl
from jax.experimental.pallas import tpu as pltpu
from jax.experimental.pallas import tpu_sc as plsc

B = 16384
D = 32
H1 = 64
H2 = 32
NC = 2
NS = 16
NW = NC * NS
B_PER_W = B // NW
CHUNK = 128
NCHUNK = B_PER_W // CHUNK


def _sc_gather_body(ut_hbm, it_hbm, uidx_hbm, iidx_hbm, u_out, i_out,
                    idx_v, rows_u, rows_i, sem):
    wid = lax.axis_index("s") * NC + lax.axis_index("c")
    base = wid * B_PER_W
    for c in range(NCHUNK):
        pltpu.sync_copy(uidx_hbm.at[pl.ds(base + c * CHUNK, CHUNK)],
                        idx_v.at[c])
        pltpu.sync_copy(iidx_hbm.at[pl.ds(base + c * CHUNK, CHUNK)],
                        idx_v.at[NCHUNK + c])
    cps = []
    for c in range(NCHUNK):
        cps.append(pltpu.async_copy(
            ut_hbm.at[idx_v.at[c]],
            rows_u.at[pl.ds(c * CHUNK, CHUNK)], sem))
    for c in range(NCHUNK):
        cps.append(pltpu.async_copy(
            it_hbm.at[idx_v.at[NCHUNK + c]],
            rows_i.at[pl.ds(c * CHUNK, CHUNK)], sem))
    for cp in cps:
        cp.wait()
    pltpu.sync_copy(rows_u, u_out.at[pl.ds(base, B_PER_W)])
    pltpu.sync_copy(rows_i, i_out.at[pl.ds(base, B_PER_W)])


_sc_gather = functools.partial(
    pl.kernel,
    mesh=plsc.VectorSubcoreMesh(core_axis_name="c", subcore_axis_name="s"),
    out_type=(jax.ShapeDtypeStruct((B, D), jnp.float32),
              jax.ShapeDtypeStruct((B, D), jnp.float32)),
    scratch_types=[
        pltpu.VMEM((2 * NCHUNK, CHUNK), jnp.int32),
        pltpu.VMEM((B_PER_W, D), jnp.float32),
        pltpu.VMEM((B_PER_W, D), jnp.float32),
        pltpu.SemaphoreType.DMA,
    ],
    compiler_params=pltpu.CompilerParams(use_tc_tiling_on_sc=False),
)(_sc_gather_body)


def _mlp_body(u_ref, i_ref, w1u_ref, w1i_ref, b1_ref, w2_ref, b2_ref,
              w3_ref, b3_ref, o_ref):
    x = (jnp.dot(u_ref[...], w1u_ref[...], preferred_element_type=jnp.float32)
         + jnp.dot(i_ref[...], w1i_ref[...], preferred_element_type=jnp.float32)
         + b1_ref[...])
    x = jnp.maximum(x, 0.0)
    x = jnp.dot(x, w2_ref[...], preferred_element_type=jnp.float32) + b2_ref[...]
    x = jnp.maximum(x, 0.0)
    y = jnp.dot(x, w3_ref[...], preferred_element_type=jnp.float32) + b3_ref[...]
    o_ref[...] = 4.0 / (1.0 + jnp.exp(-y)) + 1.0


def _mlp(u_emb, i_emb, w1u, w1i, b1, w2, b2, w3, b3):
    blk = 2048
    grid = (B // blk,)
    rep = lambda i: (0, 0)
    return pl.pallas_call(
        _mlp_body,
        grid=grid,
        in_specs=[
            pl.BlockSpec((blk, D), lambda i: (i, 0)),
            pl.BlockSpec((blk, D), lambda i: (i, 0)),
            pl.BlockSpec((D, H1), rep),
            pl.BlockSpec((D, H1), rep),
            pl.BlockSpec((1, H1), rep),
            pl.BlockSpec((H1, H2), rep),
            pl.BlockSpec((1, H2), rep),
            pl.BlockSpec((H2, 1), rep),
            pl.BlockSpec((1, 1), rep),
        ],
        out_specs=pl.BlockSpec((blk, 1), lambda i: (i, 0)),
        out_shape=jax.ShapeDtypeStruct((B, 1), jnp.float32),
    )(u_emb, i_emb, w1u, w1i, b1, w2, b2, w3, b3)


@jax.jit
def kernel(user_idx, item_idx, user_table, item_table, W1, b1, W2, b2, W3, b3):
    uidx = user_idx.astype(jnp.int32)
    iidx = item_idx.astype(jnp.int32)
    u_emb, i_emb = _sc_gather(user_table, item_table, uidx, iidx)
    return _mlp(u_emb, i_emb, W1[:D], W1[D:], b1.reshape(1, H1),
                W2, b2.reshape(1, H2), W3, b3.reshape(1, 1))
